# Initial kernel scaffold; baseline (speedup 1.0000x reference)
import jax
import jax.numpy as jnp
from jax.experimental import pallas as pl


def kernel(s, v, edge_index, edge_rbf, edge_cutoff, edge_vec, W1, b1, W2, b2, Wr, br):
    raise NotImplementedError("write your pallas kernel here")


# speedup vs baseline: 29.8024x; 29.8024x over previous
import functools

import jax
import jax.numpy as jnp
from jax import lax
from jax.experimental import pallas as pl
from jax.experimental.pallas import tpu as pltpu
from jax.experimental.pallas import tpu_sc as plsc

N_NODES = 10000
N_EDGES = 320000
H = 128
NUM_RBF = 20

NC, NS = 2, 16
NW = NC * NS
EPW = N_EDGES // NW
EPT = N_EDGES // NS
GC = 80
SC_CHUNK = 80

def _mesh():
    return plsc.VectorSubcoreMesh(
        core_axis_name="c", subcore_axis_name="s", num_cores=NC, num_subcores=NS)


@functools.cache
def _gather_stage():
    @functools.partial(
        pl.kernel,
        out_type=[
            jax.ShapeDtypeStruct((N_EDGES, H), jnp.float32),
            jax.ShapeDtypeStruct((N_EDGES, 3 * H), jnp.float32),
        ],
        mesh=_mesh(),
        scratch_types=[
            pltpu.VMEM((GC,), jnp.int32),
            pltpu.VMEM((GC, H), jnp.float32),
            pltpu.VMEM((GC, 3 * H), jnp.float32),
            pltpu.SemaphoreType.DMA,
            pltpu.SemaphoreType.DMA,
        ],
    )
    def gather_k(j_hbm, s_hbm, v_hbm, sj_out, vj_out, idx_v, srow_v, vrow_v,
                 sem_s, sem_v):
        wid = lax.axis_index("s") * NC + lax.axis_index("c")
        base = wid * EPW

        def body(k, carry):
            e0 = base + k * GC
            pltpu.sync_copy(j_hbm.at[pl.ds(e0, GC)], idx_v)
            cp_s = pltpu.async_copy(s_hbm.at[idx_v], srow_v, sem_s)
            cp_v = pltpu.async_copy(v_hbm.at[idx_v], vrow_v, sem_v)
            cp_s.wait()
            pltpu.sync_copy(srow_v, sj_out.at[pl.ds(e0, GC), :])
            cp_v.wait()
            pltpu.sync_copy(vrow_v, vj_out.at[pl.ds(e0, GC), :])
            return carry

        lax.fori_loop(0, EPW // GC, body, 0)

    return gather_k


_TCB = 640


def _tc_body(sj_ref, vj_ref, rbf_ref, cut_ref, vec_ref, w1_ref, b1_ref,
             w2_ref, b2_ref, wr_ref, br_ref, z_ref):
    sj = sj_ref[...]
    h = jnp.dot(sj, w1_ref[...], preferred_element_type=jnp.float32) + b1_ref[...]
    h = h * (1.0 / (1.0 + jnp.exp(-h)))
    h = jnp.dot(h, w2_ref[...], preferred_element_type=jnp.float32) + b2_ref[...]
    wt = jnp.dot(rbf_ref[...], wr_ref[...], preferred_element_type=jnp.float32)
    wt = (wt + br_ref[...]) * cut_ref[...]
    x = h * wt
    x_ss = x[:, :H]
    x_sv = x[:, H:2 * H]
    x_vv = x[:, 2 * H:]
    vj = vj_ref[...]
    vec = vec_ref[...]
    inner = (vj[:, :H] * vec[:, 0:1] + vj[:, H:2 * H] * vec[:, 1:2]
             + vj[:, 2 * H:] * vec[:, 2:3])
    u = x_sv + inner * x_vv
    z_ref[0] = x_ss
    z_ref[1] = u * vec[:, 0:1]
    z_ref[2] = u * vec[:, 1:2]
    z_ref[3] = u * vec[:, 2:3]


def _tc_stage(sj, vj, rbf, cut, vec, w1, b1, w2, b2, wr, br):
    grid = (N_EDGES // _TCB,)
    return pl.pallas_call(
        _tc_body,
        grid=grid,
        in_specs=[
            pl.BlockSpec((_TCB, H), lambda e: (e, 0)),
            pl.BlockSpec((_TCB, 3 * H), lambda e: (e, 0)),
            pl.BlockSpec((_TCB, NUM_RBF), lambda e: (e, 0)),
            pl.BlockSpec((_TCB, 1), lambda e: (e, 0)),
            pl.BlockSpec((_TCB, 3), lambda e: (e, 0)),
            pl.BlockSpec((H, H), lambda e: (0, 0)),
            pl.BlockSpec((1, H), lambda e: (0, 0)),
            pl.BlockSpec((H, 3 * H), lambda e: (0, 0)),
            pl.BlockSpec((1, 3 * H), lambda e: (0, 0)),
            pl.BlockSpec((NUM_RBF, 3 * H), lambda e: (0, 0)),
            pl.BlockSpec((1, 3 * H), lambda e: (0, 0)),
        ],
        out_specs=pl.BlockSpec((4, _TCB, H), lambda e: (0, e, 0)),
        out_shape=jax.ShapeDtypeStruct((4, N_EDGES, H), jnp.float32),
    )(sj, vj, rbf, cut, vec, w1, b1, w2, b2, wr, br)


@functools.cache
def _scatter_stage():
    @functools.partial(
        pl.kernel,
        out_type=jax.ShapeDtypeStruct((4, N_NODES, H), jnp.float32),
        mesh=_mesh(),
        scratch_types=[
            pltpu.VMEM((SC_CHUNK,), jnp.int32),
            pltpu.VMEM((SC_CHUNK, H), jnp.float32),
            pltpu.VMEM_SHARED((N_NODES, H), jnp.float32),
        ],
    )
    def scatter_k(i_hbm, z_hbm, zero_hbm, out4, idx_v, row_v, table):
        core = lax.axis_index("c")
        sub = lax.axis_index("s")
        for q in range(2):
            p = 2 * core + q

            @pl.when(sub == 0)
            def _zero():
                pltpu.sync_copy(zero_hbm, table)

            plsc.subcore_barrier()

            def body(k, carry):
                e0 = sub * EPT + k * SC_CHUNK
                pltpu.sync_copy(i_hbm.at[pl.ds(e0, SC_CHUNK)], idx_v)
                pltpu.sync_copy(z_hbm.at[p, pl.ds(e0, SC_CHUNK), :], row_v)
                pltpu.sync_copy(row_v, table.at[idx_v], add=True)
                return carry

            lax.fori_loop(0, EPT // SC_CHUNK, body, 0)
            plsc.subcore_barrier()

            @pl.when(sub == 0)
            def _flush():
                pltpu.sync_copy(table, out4.at[p])

            plsc.subcore_barrier()

    return scatter_k


def kernel(s, v, edge_index, edge_rbf, edge_cutoff, edge_vec, W1, b1, W2, b2,
           Wr, br):
    i = edge_index[0].astype(jnp.int32)
    j = edge_index[1].astype(jnp.int32)
    n = s.shape[0]
    v2d = v.reshape(n, 3 * H)

    sj, vj = _gather_stage()(j, s, v2d)
    z = _tc_stage(sj, vj, edge_rbf, edge_cutoff[:, None], edge_vec,
                  W1, b1[None, :], W2, b2[None, :], Wr, br[None, :])
    zero = jnp.zeros((n, H), jnp.float32)
    out4 = _scatter_stage()(i, z, zero)
    ds = out4[0]
    dv = jnp.transpose(out4[1:4], (1, 0, 2))
    return ds, dv
